# baseline (speedup 1.0000x reference)
import functools
import math

import jax
import jax.numpy as jnp
from jax import lax
from jax.experimental import pallas as pl
from jax.experimental.pallas import tpu as pltpu, tpu_sc as plsc

D_MODEL = 64
SCALE = math.sqrt(D_MODEL)
L = 16


def _mesh_info():
    info = plsc.get_sparse_core_info()
    mesh = plsc.VectorSubcoreMesh(core_axis_name="c", subcore_axis_name="s")
    return info, mesh, info.num_cores * info.num_subcores


@functools.partial(jax.jit, static_argnames=("rows",))
def _pack(table, rows=200):
    V, D = table.shape
    info, mesh, nw = _mesh_info()
    n_chunks = V // rows
    assert rows % 8 == 0 and V % rows == 0

    @functools.partial(
        pl.kernel,
        mesh=mesh,
        out_type=jax.ShapeDtypeStruct((V, 2 * D), jnp.float32),
        scratch_types=[
            pltpu.VMEM((2, rows, D), jnp.float32),
            pltpu.VMEM((2, rows, 2 * D), jnp.float32),
            pltpu.SemaphoreType.DMA((2,)),
            pltpu.SemaphoreType.DMA((2,)),
        ],
    )
    def k(table_hbm, t2_hbm, ibuf, obuf, in_sem, out_sem):
        wid = lax.axis_index("s") * info.num_cores + lax.axis_index("c")
        my_n = (n_chunks - wid + nw - 1) // nw

        def chunk_of(s):
            return s * nw + wid

        def read(s, b):
            pltpu.async_copy(
                table_hbm.at[pl.ds(chunk_of(s) * rows, rows), :],
                ibuf.at[b],
                in_sem.at[b],
            )

        def wait_read(b):
            pltpu.make_async_copy(
                table_hbm.at[pl.ds(0, rows), :], ibuf.at[b], in_sem.at[b]
            ).wait()

        def write(s, b):
            pltpu.async_copy(
                obuf.at[b],
                t2_hbm.at[pl.ds(chunk_of(s) * rows, rows), :],
                out_sem.at[b],
            )

        def wait_write(b):
            pltpu.make_async_copy(
                obuf.at[b],
                t2_hbm.at[pl.ds(0, rows), :],
                out_sem.at[b],
            ).wait()

        @pl.when(my_n >= 1)
        def _():
            read(0, 0)

        def pair(i, _):
            for b in range(2):
                s = 2 * i + b

                @pl.when(s < my_n)
                def _():
                    wait_read(b)

                    @pl.when(s + 1 < my_n)
                    def _():
                        read(s + 1, 1 - b)

                    @pl.when(s >= 2)
                    def _():
                        wait_write(b)

                    def scale_row(r, _):
                        for j in range(D // L):
                            sl = pl.ds(j * L, L)
                            obuf[b, r, sl] = ibuf[b, r, sl] * SCALE
                        return 0

                    lax.fori_loop(0, rows, scale_row, 0, unroll=4)
                    write(s, b)

            return 0

        lax.fori_loop(0, (n_chunks // nw + 2) // 2, pair, 0)

        @pl.when(my_n >= 1)
        def _():
            wait_write(0)

        @pl.when(my_n >= 2)
        def _():
            wait_write(1)

    return k(table)


@functools.partial(jax.jit, static_argnames=("chunk",))
def _gather(t2, idx, chunk=200):
    B = idx.shape[0]
    V, D2 = t2.shape
    D = D2 // 2
    info, mesh, nw = _mesh_info()
    assert B % (nw * chunk) == 0
    b_per_w = B // nw
    n_chunks = b_per_w // chunk
    assert n_chunks % 2 == 0

    @functools.partial(
        pl.kernel,
        mesh=mesh,
        out_type=jax.ShapeDtypeStruct((B, D), jnp.float32),
        scratch_types=[
            pltpu.VMEM((b_per_w,), jnp.int32),
            pltpu.VMEM((2, chunk, D2), jnp.float32),
            pltpu.VMEM((2, chunk, D), jnp.float32),
            pltpu.SemaphoreType.DMA((2,)),
            pltpu.SemaphoreType.DMA((2,)),
        ],
    )
    def k(t2_hbm, idx_hbm, out_hbm, idx_v, rows_v, cbuf, gsem, ssem):
        wid = lax.axis_index("s") * info.num_cores + lax.axis_index("c")
        base = wid * b_per_w

        pltpu.sync_copy(idx_hbm.at[pl.ds(base, b_per_w)], idx_v)

        def gather(g, b):
            pltpu.async_copy(
                t2_hbm.at[idx_v.at[pl.ds(g * chunk, chunk)]],
                rows_v.at[b],
                gsem.at[b],
            )

        def wait_gather(b):
            pltpu.make_async_copy(
                t2_hbm.at[idx_v.at[pl.ds(0, chunk)]], rows_v.at[b], gsem.at[b]
            ).wait()

        def scatter(g, b):
            pltpu.async_copy(
                cbuf.at[b],
                out_hbm.at[pl.ds(base + g * chunk, chunk)],
                ssem.at[b],
            )

        def wait_scatter(b):
            pltpu.make_async_copy(
                cbuf.at[b],
                out_hbm.at[pl.ds(base, chunk)],
                ssem.at[b],
            ).wait()

        gather(0, 0)

        def pair(i, _):
            for b in range(2):
                g = 2 * i + b
                bn = 1 - b
                wait_gather(b)

                @pl.when(g + 1 < n_chunks)
                def _():
                    gather(g + 1, bn)

                @pl.when(g >= 2)
                def _():
                    wait_scatter(b)

                def compact_row(r, _):
                    for j in range(D // L):
                        sl = pl.ds(j * L, L)
                        cbuf[b, r, sl] = rows_v[b, r, sl]
                    return 0

                lax.fori_loop(0, chunk, compact_row, 0, unroll=4)
                scatter(g, b)
            return 0

        lax.fori_loop(0, n_chunks // 2, pair, 0)
        wait_scatter((n_chunks - 2) % 2)
        wait_scatter((n_chunks - 1) % 2)

    return k(t2, idx)


def kernel(x, table):
    B0, B1 = x.shape
    idx = x.reshape(B0 * B1).astype(jnp.int32)
    t2 = _pack(table)
    out = _gather(t2, idx)
    return out.reshape(B0, B1, D_MODEL)
